# baseline (speedup 1.0000x reference)
import functools

import jax
import jax.numpy as jnp
from jax import lax
from jax.experimental import pallas as pl
from jax.experimental.pallas import tpu as pltpu
from jax.experimental.pallas import tpu_sc as plsc

B, S, D = 4, 4096, 1024
NUM_EMB = 4

ROWS = B * S
SC_ROWS = 4096
TC_ROWS = ROWS - SC_ROWS

NW = 32
SC_ROWS_PER_TILE = SC_ROWS // NW
BUF_ROWS = 64
N_WRITES = SC_ROWS_PER_TILE // BUF_ROWS

TC_BLK = 2048


def _sc_body(idx_hbm, table_hbm, out_hbm, idx_v, buf, shared, gsem, wsem):
    sid = lax.axis_index("s")
    wid = sid * 2 + lax.axis_index("c")
    base = wid * SC_ROWS_PER_TILE

    @pl.when(sid == 0)
    def _stage():
        pltpu.sync_copy(idx_hbm, idx_v)
        pltpu.async_copy(table_hbm.at[idx_v], buf, gsem).wait()
        for k in range(BUF_ROWS // 8):
            pltpu.sync_copy(buf, shared.at[pl.ds(8 * k, 8)])

    plsc.subcore_barrier()
    copies = [
        pltpu.async_copy(
            shared, out_hbm.at[pl.ds(base + j * BUF_ROWS, BUF_ROWS)], wsem
        )
        for j in range(N_WRITES)
    ]
    for c in copies:
        c.wait()


@functools.partial(
    pl.kernel,
    out_type=jax.ShapeDtypeStruct((SC_ROWS, D), jnp.float32),
    mesh=plsc.VectorSubcoreMesh(core_axis_name="c", subcore_axis_name="s"),
    scratch_types=[
        pltpu.VMEM((8,), jnp.int32),
        pltpu.VMEM((8, D), jnp.float32),
        pltpu.VMEM_SHARED((BUF_ROWS, D), jnp.float32),
        pltpu.SemaphoreType.DMA,
        pltpu.SemaphoreType.DMA,
    ],
)
def _sc_broadcast(idx_hbm, table_hbm, out_hbm, idx_v, buf, shared, gsem, wsem):
    _sc_body(idx_hbm, table_hbm, out_hbm, idx_v, buf, shared, gsem, wsem)


def _tc_kernel(idx_ref, table_ref, out_ref):
    idx = idx_ref[0]
    row_ids = jax.lax.broadcasted_iota(jnp.int32, (NUM_EMB, D), 0)
    row = jnp.sum(jnp.where(row_ids == idx, table_ref[...], 0.0),
                  axis=0, keepdims=True)
    out_ref[...] = jnp.broadcast_to(row, out_ref.shape)


def kernel(x, modality_index, emb_table):
    del x
    idx = jnp.asarray(modality_index, jnp.int32)
    idx_vec = jnp.full((8,), idx, dtype=jnp.int32)

    sc_out = _sc_broadcast(idx_vec, emb_table)

    tc_out = pl.pallas_call(
        _tc_kernel,
        grid_spec=pltpu.PrefetchScalarGridSpec(
            num_scalar_prefetch=1,
            grid=(TC_ROWS // TC_BLK,),
            in_specs=[pl.BlockSpec((NUM_EMB, D), lambda i, *_: (0, 0))],
            out_specs=pl.BlockSpec((TC_BLK, D), lambda i, *_: (i, 0)),
        ),
        out_shape=jax.ShapeDtypeStruct((TC_ROWS, D), jnp.float32),
    )(idx.reshape((1,)), emb_table)

    return jnp.concatenate([tc_out, sc_out], axis=0).reshape(B, S, D)


# speedup vs baseline: 3.6932x; 3.6932x over previous
import jax
import jax.numpy as jnp
from jax.experimental import pallas as pl
from jax.experimental.pallas import tpu as pltpu

B, S, D = 4, 4096, 1024
NUM_EMB = 4

ROWS = B * S
BLK = 1024
N_COPIES = ROWS // BLK


def _bcast_kernel(idx_ref, table_ref, out_ref, blk, sem):
    idx = idx_ref[0]
    row_ids = jax.lax.broadcasted_iota(jnp.int32, (NUM_EMB, D), 0)
    row = jnp.sum(jnp.where(row_ids == idx, table_ref[...], 0.0),
                  axis=0, keepdims=True)
    blk[...] = jnp.broadcast_to(row, blk.shape)
    copies = [
        pltpu.make_async_copy(blk, out_ref.at[pl.ds(i * BLK, BLK), :], sem)
        for i in range(N_COPIES)
    ]
    for c in copies:
        c.start()
    for c in copies:
        c.wait()


def kernel(x, modality_index, emb_table):
    del x
    idx = jnp.asarray(modality_index, jnp.int32).reshape((1,))
    out = pl.pallas_call(
        _bcast_kernel,
        grid_spec=pltpu.PrefetchScalarGridSpec(
            num_scalar_prefetch=1,
            grid=(1,),
            in_specs=[pl.BlockSpec((NUM_EMB, D), lambda i, *_: (0, 0))],
            out_specs=pl.BlockSpec(memory_space=pl.ANY),
            scratch_shapes=[
                pltpu.VMEM((BLK, D), jnp.float32),
                pltpu.SemaphoreType.DMA,
            ],
        ),
        out_shape=jax.ShapeDtypeStruct((ROWS, D), jnp.float32),
    )(idx, emb_table)
    return out.reshape(B, S, D)


# speedup vs baseline: 3.7093x; 1.0043x over previous
import jax
import jax.numpy as jnp
from jax.experimental import pallas as pl
from jax.experimental.pallas import tpu as pltpu

B, S, D = 4, 4096, 1024
NUM_EMB = 4

ROWS = B * S
BLK = 512
N_COPIES = ROWS // BLK


def _bcast_kernel(idx_ref, table_ref, out_ref, blk, sem):
    idx = idx_ref[0]
    row_ids = jax.lax.broadcasted_iota(jnp.int32, (NUM_EMB, D), 0)
    row = jnp.sum(jnp.where(row_ids == idx, table_ref[...], 0.0),
                  axis=0, keepdims=True)
    blk[...] = jnp.broadcast_to(row, blk.shape)
    copies = [
        pltpu.make_async_copy(blk, out_ref.at[pl.ds(i * BLK, BLK), :], sem)
        for i in range(N_COPIES)
    ]
    for c in copies:
        c.start()
    for c in copies:
        c.wait()


def kernel(x, modality_index, emb_table):
    del x
    idx = jnp.asarray(modality_index, jnp.int32).reshape((1,))
    out = pl.pallas_call(
        _bcast_kernel,
        grid_spec=pltpu.PrefetchScalarGridSpec(
            num_scalar_prefetch=1,
            grid=(1,),
            in_specs=[pl.BlockSpec((NUM_EMB, D), lambda i, *_: (0, 0))],
            out_specs=pl.BlockSpec(memory_space=pl.ANY),
            scratch_shapes=[
                pltpu.VMEM((BLK, D), jnp.float32),
                pltpu.SemaphoreType.DMA,
            ],
        ),
        out_shape=jax.ShapeDtypeStruct((ROWS, D), jnp.float32),
    )(idx, emb_table)
    return out.reshape(B, S, D)
